# baseline (speedup 1.0000x reference)
import functools

import jax
import jax.numpy as jnp
from jax import lax
from jax.experimental import pallas as pl
from jax.experimental.pallas import tpu as pltpu
from jax.experimental.pallas import tpu_sc as plsc

XS = 400
NCELL = XS * XS
PLANE = 163840
DUMP = NCELL
BATCH = 4
FEAT = 64
NPAD = 12288
CHUNK = NPAD // 16
NROWS = CHUNK // 128
EXP = NCELL // 16
ZB = 1024
NB = BATCH // 2

_mesh = plsc.VectorSubcoreMesh(core_axis_name="c", subcore_axis_name="s")


@functools.partial(
    pl.kernel,
    out_type=jax.ShapeDtypeStruct((BATCH * FEAT * NCELL,), jnp.float32),
    scratch_types=[
        pltpu.VMEM_SHARED((PLANE,), jnp.float32),
        pltpu.VMEM((NB, NROWS, 128), jnp.int32),
        pltpu.VMEM((CHUNK,), jnp.float32),
        pltpu.VMEM((ZB,), jnp.float32),
        pltpu.VMEM((CHUNK,), jnp.int32),
        pltpu.VMEM((CHUNK,), jnp.int32),
        pltpu.VMEM((CHUNK,), jnp.int32),
        pltpu.VMEM((EXP,), jnp.float32),
    ],
    mesh=_mesh,
)
def _scatter_planes(vals_hbm, y_hbm, x_hbm, m_hbm, out_hbm,
                    plane_sh, idx_v, vals_v, zeros_v, y_v, x_v, m_v, exp_v):
    c = lax.axis_index("c")
    s = lax.axis_index("s")
    base = s * CHUNK

    zv = jnp.zeros((16,), jnp.float32)
    for i in range(ZB // 16):
        zeros_v[pl.ds(i * 16, 16)] = zv

    stripe = PLANE // 16
    for i in range(stripe // ZB):
        pltpu.sync_copy(zeros_v, plane_sh.at[pl.ds(s * stripe + i * ZB, ZB)])

    lane = lax.iota(jnp.int32, 16)
    for bl in range(NB):
        b = c * NB + bl
        pltpu.sync_copy(y_hbm.at[pl.ds(b * NPAD + base, CHUNK)], y_v)
        pltpu.sync_copy(x_hbm.at[pl.ds(b * NPAD + base, CHUNK)], x_v)
        pltpu.sync_copy(m_hbm.at[pl.ds(b * NPAD + base, CHUNK)], m_v)
        for i in range(CHUNK // 16):
            yv = y_v[pl.ds(i * 16, 16)]
            xv = x_v[pl.ds(i * 16, 16)]
            mv = m_v[pl.ds(i * 16, 16)]
            dump = DUMP + ((lane + i * 16) & 1023)
            iv = jnp.where(mv == 1, yv * XS + xv, dump)
            idx_v[bl, i // 8, pl.ds((i % 8) * 16, 16)] = iv

    plsc.subcore_barrier()

    for bl in range(NB):
        b = c * NB + bl

        def plane_body(f, carry, bl=bl, b=b):
            plane_id = b * FEAT + f
            pltpu.sync_copy(vals_hbm.at[pl.ds(plane_id * NPAD + base, CHUNK)],
                            vals_v)
            for j in range(NROWS):
                pltpu.sync_copy(vals_v.at[pl.ds(j * 128, 128)],
                                plane_sh.at[idx_v.at[bl, j]], add=True)
            plsc.subcore_barrier()
            pltpu.sync_copy(plane_sh.at[pl.ds(s * EXP, EXP)], exp_v)
            pltpu.sync_copy(exp_v,
                            out_hbm.at[pl.ds(plane_id * NCELL + s * EXP, EXP)])
            plsc.subcore_barrier()
            for j in range(NROWS):
                pltpu.sync_copy(zeros_v.at[pl.ds(j * 128, 128)],
                                plane_sh.at[idx_v.at[bl, j]])
            plsc.subcore_barrier()
            return carry

        lax.fori_loop(0, FEAT, plane_body, 0)


def kernel(pillars, coord, contains_pillars):
    batch, n_pillars, _ = pillars.shape
    pad = NPAD - n_pillars
    vals = jnp.transpose(pillars.astype(jnp.float32), (0, 2, 1))
    vals = jnp.pad(vals, ((0, 0), (0, 0), (0, pad))).reshape(-1)
    yc = jnp.pad(coord[:, :, 1].astype(jnp.int32), ((0, 0), (0, pad))).reshape(-1)
    xc = jnp.pad(coord[:, :, 2].astype(jnp.int32), ((0, 0), (0, pad))).reshape(-1)
    mc = jnp.pad(contains_pillars.astype(jnp.int32), ((0, 0), (0, pad))).reshape(-1)
    out = _scatter_planes(vals, yc, xc, mc)
    return out.reshape(batch, FEAT, XS, XS)


# speedup vs baseline: 1.5568x; 1.5568x over previous
import functools

import jax
import jax.numpy as jnp
from jax import lax
from jax.experimental import pallas as pl
from jax.experimental.pallas import tpu as pltpu
from jax.experimental.pallas import tpu_sc as plsc

XS = 400
NTY = 50
NTYP = 64
STRIPE_W = 4096
DUMP = NTY * STRIPE_W
PLANE = NTYP * STRIPE_W
BATCH = 4
FEAT = 64
NPAD = 12288
CHUNK = NPAD // 16
NROWS = CHUNK // 128
ZB = 1024
NB = BATCH // 2

_mesh = plsc.VectorSubcoreMesh(core_axis_name="c", subcore_axis_name="s")


@functools.partial(
    pl.kernel,
    out_type=jax.ShapeDtypeStruct((BATCH, FEAT, XS, XS), jnp.float32),
    scratch_types=[
        pltpu.VMEM_SHARED((PLANE,), jnp.float32),
        pltpu.VMEM_SHARED((PLANE,), jnp.float32),
        pltpu.VMEM((NB, NROWS, 128), jnp.int32),
        pltpu.VMEM((CHUNK,), jnp.float32),
        pltpu.VMEM((ZB,), jnp.float32),
        pltpu.VMEM((CHUNK,), jnp.int32),
        pltpu.VMEM((CHUNK,), jnp.int32),
        pltpu.VMEM((CHUNK,), jnp.int32),
        pltpu.VMEM((STRIPE_W,), jnp.float32),
        pltpu.VMEM((STRIPE_W,), jnp.float32),
        pltpu.VMEM((8, 384), jnp.float32),
        pltpu.VMEM((8, 384), jnp.float32),
        pltpu.VMEM((8, 16), jnp.float32),
        pltpu.VMEM((8, 16), jnp.float32),
        pltpu.SemaphoreType.DMA,
        pltpu.SemaphoreType.DMA,
        pltpu.SemaphoreType.DMA,
        pltpu.SemaphoreType.DMA,
        pltpu.SemaphoreType.DMA,
        pltpu.SemaphoreType.DMA,
    ],
    mesh=_mesh,
)
def _scatter_planes(vals_hbm, y_hbm, x_hbm, m_hbm, out_hbm,
                    plane_a, plane_b, idx_v, vals_v, zeros_v, y_v, x_v, m_v,
                    gf_a, gf_b, g2d_a, g2d_b, g2dt_a, g2dt_b,
                    sem_s, sem_f0, sem_f1, sem_e0, sem_e1, sem_r):
    planes = (plane_a, plane_b)
    g2ds = (g2d_a, g2d_b)
    g2dts = (g2dt_a, g2dt_b)
    semf = (sem_f0, sem_f1)
    seme = (sem_e0, sem_e1)
    c = lax.axis_index("c")
    s = lax.axis_index("s")
    base = s * CHUNK
    gfs = (gf_a, gf_b)

    zv = jnp.zeros((16,), jnp.float32)
    for i in range(ZB // 16):
        zeros_v[pl.ds(i * 16, 16)] = zv

    tile_words = PLANE // 16
    for p in range(2):
        for i in range(tile_words // ZB):
            pltpu.sync_copy(
                zeros_v,
                planes[p].at[pl.ds(s * tile_words + i * ZB, ZB)])

    lane = lax.iota(jnp.int32, 16)
    for bl in range(NB):
        b = c * NB + bl
        pltpu.sync_copy(y_hbm.at[pl.ds(b * NPAD + base, CHUNK)], y_v)
        pltpu.sync_copy(x_hbm.at[pl.ds(b * NPAD + base, CHUNK)], x_v)
        pltpu.sync_copy(m_hbm.at[pl.ds(b * NPAD + base, CHUNK)], m_v)
        for i in range(CHUNK // 16):
            yv = y_v[pl.ds(i * 16, 16)]
            xv = x_v[pl.ds(i * 16, 16)]
            mv = m_v[pl.ds(i * 16, 16)]
            addr = (((yv >> 3) << 12) + ((xv >> 7) << 10)
                    + ((yv & 7) << 7) + (xv & 127))
            dump = DUMP + ((lane + i * 16) & 1023)
            iv = jnp.where(mv == 1, addr, dump)
            idx_v[bl, i // 8, pl.ds((i % 8) * 16, 16)] = iv

    plsc.subcore_barrier()

    def issue_scatter(pbuf, bl, f):
        voff = (f * BATCH + (c * NB + bl)) * NPAD + base
        pltpu.sync_copy(vals_hbm.at[pl.ds(voff, CHUNK)], vals_v)
        return [
            pltpu.async_copy(vals_v.at[pl.ds(j * 128, 128)],
                             pbuf.at[idx_v.at[bl, j]], sem_s, add=True)
            for j in range(NROWS)
        ]

    def export_plane(pbuf, bl, fp):
        b = c * NB + bl
        fd = [None] * 4
        fd[0] = pltpu.async_copy(pbuf.at[pl.ds(s * STRIPE_W, STRIPE_W)],
                                 gfs[0], semf[0])
        wds = {}
        for i in range(4):
            ty = i * 16 + s
            if i < 3:
                nty = (i + 1) * 16 + s
                fd[i + 1] = pltpu.async_copy(
                    pbuf.at[pl.ds(nty * STRIPE_W, STRIPE_W)],
                    gfs[(i + 1) % 2], semf[(i + 1) % 2])
            fd[i].wait()
            gf = gfs[i % 2]
            g2d = g2ds[i % 2]
            g2dt = g2dts[i % 2]
            if i >= 2:
                for d in wds.pop(i - 2):
                    d.wait()

            def emit(ty=ty, gf=gf, g2d=g2d, g2dt=g2dt, i=i):
                for r in range(8):
                    for tx in range(3):
                        for u in range(8):
                            g2d[r, pl.ds(tx * 128 + u * 16, 16)] = (
                                gf[pl.ds(tx * 1024 + r * 128 + u * 16, 16)])
                    g2dt[r, pl.ds(0, 16)] = gf[pl.ds(3 * 1024 + r * 128, 16)]
                yo = pl.multiple_of(ty * 8, 8)
                d1 = pltpu.async_copy(
                    g2d, out_hbm.at[b, fp, pl.ds(yo, 8), pl.ds(0, 384)],
                    seme[i % 2])
                d2 = pltpu.async_copy(
                    g2dt, out_hbm.at[b, fp, pl.ds(yo, 8), pl.ds(384, 16)],
                    seme[i % 2])
                if i < 3:
                    wds[i] = (d1, d2)
                else:
                    d1.wait()
                    d2.wait()

            if i < 3:
                emit()
            else:
                pl.when(ty < NTY)(emit)
        for d in wds.pop(2):
            d.wait()

    def restore_plane(pbuf, bl):
        rds = [
            pltpu.async_copy(zeros_v.at[pl.ds(j * 128, 128)],
                             pbuf.at[idx_v.at[bl, j]], sem_r)
            for j in range(NROWS)
        ]
        for d in rds:
            d.wait()

    for bl in range(NB):
        def plane_loop(f, carry, bl=bl):
            sds = issue_scatter(planes[0], bl, f)
            for d in sds:
                d.wait()
            plsc.subcore_barrier()
            export_plane(planes[0], bl, f)
            plsc.subcore_barrier()
            restore_plane(planes[0], bl)
            plsc.subcore_barrier()
            return carry

        lax.fori_loop(0, FEAT, plane_loop, 0)


def kernel(pillars, coord, contains_pillars):
    batch, n_pillars, _ = pillars.shape
    pad = NPAD - n_pillars
    vals = jnp.transpose(pillars.astype(jnp.float32), (2, 0, 1))
    vals = jnp.pad(vals, ((0, 0), (0, 0), (0, pad))).reshape(-1)
    yc = jnp.pad(coord[:, :, 1].astype(jnp.int32), ((0, 0), (0, pad))).reshape(-1)
    xc = jnp.pad(coord[:, :, 2].astype(jnp.int32), ((0, 0), (0, pad))).reshape(-1)
    mc = jnp.pad(contains_pillars.astype(jnp.int32), ((0, 0), (0, pad))).reshape(-1)
    return _scatter_planes(vals, yc, xc, mc)


# speedup vs baseline: 1.6378x; 1.0520x over previous
import functools

import jax
import jax.numpy as jnp
from jax import lax
from jax.experimental import pallas as pl
from jax.experimental.pallas import tpu as pltpu
from jax.experimental.pallas import tpu_sc as plsc

XS = 400
NTY = 50
NTYP = 64
STRIPE_W = 4096
DUMP = NTY * STRIPE_W
PLANE = NTYP * STRIPE_W
BATCH = 4
FEAT = 64
NPAD = 12288
CHUNK = NPAD // 16
NROWS = CHUNK // 128
ZB = 1024
NB = BATCH // 2

_mesh = plsc.VectorSubcoreMesh(core_axis_name="c", subcore_axis_name="s")


@functools.partial(
    pl.kernel,
    out_type=jax.ShapeDtypeStruct((BATCH, FEAT, XS, XS), jnp.float32),
    scratch_types=[
        pltpu.VMEM_SHARED((PLANE,), jnp.float32),
        pltpu.VMEM_SHARED((PLANE,), jnp.float32),
        pltpu.VMEM((NB, NROWS, 128), jnp.int32),
        pltpu.VMEM((CHUNK,), jnp.float32),
        pltpu.VMEM((ZB,), jnp.float32),
        pltpu.VMEM((CHUNK,), jnp.int32),
        pltpu.VMEM((CHUNK,), jnp.int32),
        pltpu.VMEM((CHUNK,), jnp.int32),
        pltpu.VMEM((STRIPE_W,), jnp.float32),
        pltpu.VMEM((STRIPE_W,), jnp.float32),
        pltpu.VMEM((8, 384), jnp.float32),
        pltpu.VMEM((8, 384), jnp.float32),
        pltpu.VMEM((8, 16), jnp.float32),
        pltpu.VMEM((8, 16), jnp.float32),
        pltpu.SemaphoreType.DMA,
        pltpu.SemaphoreType.DMA,
        pltpu.SemaphoreType.DMA,
        pltpu.SemaphoreType.DMA,
        pltpu.SemaphoreType.DMA,
        pltpu.SemaphoreType.DMA,
    ],
    mesh=_mesh,
)
def _scatter_planes(vals_hbm, y_hbm, x_hbm, m_hbm, out_hbm,
                    plane_a, plane_b, idx_v, vals_v, zeros_v, y_v, x_v, m_v,
                    gf_a, gf_b, g2d_a, g2d_b, g2dt_a, g2dt_b,
                    sem_s, sem_f0, sem_f1, sem_e0, sem_e1, sem_r):
    planes = (plane_a, plane_b)
    g2ds = (g2d_a, g2d_b)
    g2dts = (g2dt_a, g2dt_b)
    semf = (sem_f0, sem_f1)
    seme = (sem_e0, sem_e1)
    c = lax.axis_index("c")
    s = lax.axis_index("s")
    base = s * CHUNK
    gfs = (gf_a, gf_b)

    zv = jnp.zeros((16,), jnp.float32)
    for i in range(ZB // 16):
        zeros_v[pl.ds(i * 16, 16)] = zv

    tile_words = PLANE // 16
    for p in range(2):
        for i in range(tile_words // ZB):
            pltpu.sync_copy(
                zeros_v,
                planes[p].at[pl.ds(s * tile_words + i * ZB, ZB)])

    lane = lax.iota(jnp.int32, 16)
    for bl in range(NB):
        b = c * NB + bl
        pltpu.sync_copy(y_hbm.at[pl.ds(b * NPAD + base, CHUNK)], y_v)
        pltpu.sync_copy(x_hbm.at[pl.ds(b * NPAD + base, CHUNK)], x_v)
        pltpu.sync_copy(m_hbm.at[pl.ds(b * NPAD + base, CHUNK)], m_v)
        for i in range(CHUNK // 16):
            yv = y_v[pl.ds(i * 16, 16)]
            xv = x_v[pl.ds(i * 16, 16)]
            mv = m_v[pl.ds(i * 16, 16)]
            addr = (((yv >> 3) << 12) + ((xv >> 7) << 10)
                    + ((yv & 7) << 7) + (xv & 127))
            dump = DUMP + ((lane + i * 16) & 1023)
            iv = jnp.where(mv == 1, addr, dump)
            idx_v[bl, i // 8, pl.ds((i % 8) * 16, 16)] = iv

    plsc.subcore_barrier()

    def issue_scatter(pbuf, bl, f):
        voff = (f * BATCH + (c * NB + bl)) * NPAD + base
        pltpu.sync_copy(vals_hbm.at[pl.ds(voff, CHUNK)], vals_v)
        return [
            pltpu.async_copy(vals_v.at[pl.ds(j * 128, 128)],
                             pbuf.at[idx_v.at[bl, j]], sem_s, add=True)
            for j in range(NROWS)
        ]

    def export_plane(pbuf, bl, fp):
        b = c * NB + bl
        fd = [None] * 4
        fd[0] = pltpu.async_copy(pbuf.at[pl.ds(s * STRIPE_W, STRIPE_W)],
                                 gfs[0], semf[0])
        wds = {}
        for i in range(4):
            ty = i * 16 + s
            if i < 3:
                nty = (i + 1) * 16 + s
                fd[i + 1] = pltpu.async_copy(
                    pbuf.at[pl.ds(nty * STRIPE_W, STRIPE_W)],
                    gfs[(i + 1) % 2], semf[(i + 1) % 2])
            fd[i].wait()
            gf = gfs[i % 2]
            g2d = g2ds[i % 2]
            g2dt = g2dts[i % 2]
            if i >= 2:
                for d in wds.pop(i - 2):
                    d.wait()

            def emit(ty=ty, gf=gf, g2d=g2d, g2dt=g2dt, i=i):
                for r in range(8):
                    for tx in range(3):
                        for u in range(8):
                            g2d[r, pl.ds(tx * 128 + u * 16, 16)] = (
                                gf[pl.ds(tx * 1024 + r * 128 + u * 16, 16)])
                    g2dt[r, pl.ds(0, 16)] = gf[pl.ds(3 * 1024 + r * 128, 16)]
                yo = pl.multiple_of(ty * 8, 8)
                d1 = pltpu.async_copy(
                    g2d, out_hbm.at[b, fp, pl.ds(yo, 8), pl.ds(0, 384)],
                    seme[i % 2])
                d2 = pltpu.async_copy(
                    g2dt, out_hbm.at[b, fp, pl.ds(yo, 8), pl.ds(384, 16)],
                    seme[i % 2])
                if i < 3:
                    wds[i] = (d1, d2)
                else:
                    d1.wait()
                    d2.wait()

            if i < 3:
                emit()
            else:
                pl.when(ty < NTY)(emit)
        for d in wds.pop(2):
            d.wait()

    def restore_plane(pbuf, bl):
        rds = [
            pltpu.async_copy(zeros_v.at[pl.ds(j * 128, 128)],
                             pbuf.at[idx_v.at[bl, j]], sem_r)
            for j in range(NROWS)
        ]
        for d in rds:
            d.wait()

    for bl in range(NB):
        def pair_body(g, carry, bl=bl):
            for p in range(2):
                f = g * 2 + p
                pbuf = planes[p]
                qbuf = planes[1 - p]
                sds = issue_scatter(pbuf, bl, f)

                @pl.when(f >= 1)
                def _(qbuf=qbuf, bl=bl, f=f):
                    export_plane(qbuf, bl, f - 1)

                plsc.subcore_barrier()

                @pl.when(f >= 1)
                def _(qbuf=qbuf, bl=bl):
                    restore_plane(qbuf, bl)

                for d in sds:
                    d.wait()
                plsc.subcore_barrier()
            return carry

        lax.fori_loop(0, FEAT // 2, pair_body, 0)
        export_plane(planes[1], bl, FEAT - 1)
        plsc.subcore_barrier()
        restore_plane(planes[1], bl)
        plsc.subcore_barrier()


def kernel(pillars, coord, contains_pillars):
    batch, n_pillars, _ = pillars.shape
    pad = NPAD - n_pillars
    vals = jnp.transpose(pillars.astype(jnp.float32), (2, 0, 1))
    vals = jnp.pad(vals, ((0, 0), (0, 0), (0, pad))).reshape(-1)
    yc = jnp.pad(coord[:, :, 1].astype(jnp.int32), ((0, 0), (0, pad))).reshape(-1)
    xc = jnp.pad(coord[:, :, 2].astype(jnp.int32), ((0, 0), (0, pad))).reshape(-1)
    mc = jnp.pad(contains_pillars.astype(jnp.int32), ((0, 0), (0, pad))).reshape(-1)
    return _scatter_planes(vals, yc, xc, mc)


# speedup vs baseline: 1.6536x; 1.0097x over previous
import functools

import jax
import jax.numpy as jnp
from jax import lax
from jax.experimental import pallas as pl
from jax.experimental.pallas import tpu as pltpu
from jax.experimental.pallas import tpu_sc as plsc

XS = 400
NTY = 50
NTYP = 64
STRIPE_W = 4096
DUMP = NTY * STRIPE_W
PLANE = NTYP * STRIPE_W
BATCH = 4
FEAT = 64
NPAD = 12288
CHUNK = NPAD // 16
NROWS = CHUNK // 128
ZB = 1024
NB = BATCH // 2

_mesh = plsc.VectorSubcoreMesh(core_axis_name="c", subcore_axis_name="s")


@functools.partial(
    pl.kernel,
    out_type=jax.ShapeDtypeStruct((BATCH, FEAT, XS, XS), jnp.float32),
    scratch_types=[
        pltpu.VMEM_SHARED((PLANE,), jnp.float32),
        pltpu.VMEM_SHARED((PLANE,), jnp.float32),
        pltpu.VMEM((NB, NROWS, 128), jnp.int32),
        pltpu.VMEM((CHUNK,), jnp.float32),
        pltpu.VMEM((ZB,), jnp.float32),
        pltpu.VMEM((CHUNK,), jnp.int32),
        pltpu.VMEM((CHUNK,), jnp.int32),
        pltpu.VMEM((CHUNK,), jnp.int32),
        pltpu.VMEM((STRIPE_W,), jnp.float32),
        pltpu.VMEM((STRIPE_W,), jnp.float32),
        pltpu.VMEM((8, 384), jnp.float32),
        pltpu.VMEM((8, 384), jnp.float32),
        pltpu.VMEM((8, 16), jnp.float32),
        pltpu.VMEM((8, 16), jnp.float32),
        pltpu.VMEM((STRIPE_W,), jnp.float32),
        pltpu.SemaphoreType.DMA,
        pltpu.SemaphoreType.DMA,
        pltpu.SemaphoreType.DMA,
        pltpu.SemaphoreType.DMA,
        pltpu.SemaphoreType.DMA,
        pltpu.SemaphoreType.DMA,
    ],
    mesh=_mesh,
)
def _scatter_planes(vals_hbm, y_hbm, x_hbm, m_hbm, out_hbm,
                    plane_a, plane_b, idx_v, vals_v, zeros_v, y_v, x_v, m_v,
                    gf_a, gf_b, g2d_a, g2d_b, g2dt_a, g2dt_b, zeros4k,
                    sem_s, sem_f0, sem_f1, sem_e0, sem_e1, sem_z):
    planes = (plane_a, plane_b)
    g2ds = (g2d_a, g2d_b)
    g2dts = (g2dt_a, g2dt_b)
    semf = (sem_f0, sem_f1)
    seme = (sem_e0, sem_e1)
    c = lax.axis_index("c")
    s = lax.axis_index("s")
    base = s * CHUNK
    gfs = (gf_a, gf_b)

    zv = jnp.zeros((16,), jnp.float32)
    for i in range(ZB // 16):
        zeros_v[pl.ds(i * 16, 16)] = zv

    for i in range(STRIPE_W // 16):
        zeros4k[pl.ds(i * 16, 16)] = zv

    tile_words = PLANE // 16
    for p in range(2):
        for i in range(tile_words // ZB):
            pltpu.sync_copy(
                zeros_v,
                planes[p].at[pl.ds(s * tile_words + i * ZB, ZB)])

    lane = lax.iota(jnp.int32, 16)
    for bl in range(NB):
        b = c * NB + bl
        pltpu.sync_copy(y_hbm.at[pl.ds(b * NPAD + base, CHUNK)], y_v)
        pltpu.sync_copy(x_hbm.at[pl.ds(b * NPAD + base, CHUNK)], x_v)
        pltpu.sync_copy(m_hbm.at[pl.ds(b * NPAD + base, CHUNK)], m_v)
        for i in range(CHUNK // 16):
            yv = y_v[pl.ds(i * 16, 16)]
            xv = x_v[pl.ds(i * 16, 16)]
            mv = m_v[pl.ds(i * 16, 16)]
            addr = (((yv >> 3) << 12) + ((xv >> 7) << 10)
                    + ((yv & 7) << 7) + (xv & 127))
            dump = DUMP + ((lane + i * 16) & 1023)
            iv = jnp.where(mv == 1, addr, dump)
            idx_v[bl, i // 8, pl.ds((i % 8) * 16, 16)] = iv

    plsc.subcore_barrier()

    def issue_scatter(pbuf, bl, f):
        voff = (f * BATCH + (c * NB + bl)) * NPAD + base
        pltpu.sync_copy(vals_hbm.at[pl.ds(voff, CHUNK)], vals_v)
        return [
            pltpu.async_copy(vals_v.at[pl.ds(j * 128, 128)],
                             pbuf.at[idx_v.at[bl, j]], sem_s, add=True)
            for j in range(NROWS)
        ]

    def export_plane(pbuf, bl, fp):
        b = c * NB + bl
        fd = [None] * 4
        fd[0] = pltpu.async_copy(pbuf.at[pl.ds(s * STRIPE_W, STRIPE_W)],
                                 gfs[0], semf[0])
        wds = {}
        zds = []
        for i in range(4):
            ty = i * 16 + s
            if i < 3:
                nty = (i + 1) * 16 + s
                fd[i + 1] = pltpu.async_copy(
                    pbuf.at[pl.ds(nty * STRIPE_W, STRIPE_W)],
                    gfs[(i + 1) % 2], semf[(i + 1) % 2])
            fd[i].wait()
            zds.append(pltpu.async_copy(
                zeros4k, pbuf.at[pl.ds(ty * STRIPE_W, STRIPE_W)], sem_z))
            gf = gfs[i % 2]
            g2d = g2ds[i % 2]
            g2dt = g2dts[i % 2]
            if i >= 2:
                for d in wds.pop(i - 2):
                    d.wait()

            def emit(ty=ty, gf=gf, g2d=g2d, g2dt=g2dt, i=i):
                for r in range(8):
                    for tx in range(3):
                        for u in range(8):
                            g2d[r, pl.ds(tx * 128 + u * 16, 16)] = (
                                gf[pl.ds(tx * 1024 + r * 128 + u * 16, 16)])
                    g2dt[r, pl.ds(0, 16)] = gf[pl.ds(3 * 1024 + r * 128, 16)]
                yo = pl.multiple_of(ty * 8, 8)
                d1 = pltpu.async_copy(
                    g2d, out_hbm.at[b, fp, pl.ds(yo, 8), pl.ds(0, 384)],
                    seme[i % 2])
                d2 = pltpu.async_copy(
                    g2dt, out_hbm.at[b, fp, pl.ds(yo, 8), pl.ds(384, 16)],
                    seme[i % 2])
                if i < 3:
                    wds[i] = (d1, d2)
                else:
                    d1.wait()
                    d2.wait()

            if i < 3:
                emit()
            else:
                pl.when(ty < NTY)(emit)
        for d in wds.pop(2):
            d.wait()
        for d in zds:
            d.wait()

    for bl in range(NB):
        def pair_body(g, carry, bl=bl):
            for p in range(2):
                f = g * 2 + p
                pbuf = planes[p]
                qbuf = planes[1 - p]
                sds = issue_scatter(pbuf, bl, f)

                @pl.when(f >= 1)
                def _(qbuf=qbuf, bl=bl, f=f):
                    export_plane(qbuf, bl, f - 1)

                for d in sds:
                    d.wait()
                plsc.subcore_barrier()
            return carry

        lax.fori_loop(0, FEAT // 2, pair_body, 0)
        export_plane(planes[1], bl, FEAT - 1)
        plsc.subcore_barrier()


def kernel(pillars, coord, contains_pillars):
    batch, n_pillars, _ = pillars.shape
    pad = NPAD - n_pillars
    vals = jnp.transpose(pillars.astype(jnp.float32), (2, 0, 1))
    vals = jnp.pad(vals, ((0, 0), (0, 0), (0, pad))).reshape(-1)
    yc = jnp.pad(coord[:, :, 1].astype(jnp.int32), ((0, 0), (0, pad))).reshape(-1)
    xc = jnp.pad(coord[:, :, 2].astype(jnp.int32), ((0, 0), (0, pad))).reshape(-1)
    mc = jnp.pad(contains_pillars.astype(jnp.int32), ((0, 0), (0, pad))).reshape(-1)
    return _scatter_planes(vals, yc, xc, mc)


# speedup vs baseline: 1.7824x; 1.0779x over previous
import functools

import jax
import jax.numpy as jnp
from jax import lax
from jax.experimental import pallas as pl
from jax.experimental.pallas import tpu as pltpu
from jax.experimental.pallas import tpu_sc as plsc

XS = 400
NTY = 50
NTYP = 64
STRIPE_W = 4096
DUMP = NTY * STRIPE_W
PLANE = NTYP * STRIPE_W
BATCH = 4
FEAT = 64
NPAD = 12288
CHUNK = NPAD // 16
NROWS = CHUNK // 128
ZB = 1024
NB = BATCH // 2

_mesh = plsc.VectorSubcoreMesh(core_axis_name="c", subcore_axis_name="s")


@functools.partial(
    pl.kernel,
    out_type=jax.ShapeDtypeStruct((BATCH, FEAT, XS, XS), jnp.float32),
    scratch_types=[
        pltpu.VMEM_SHARED((PLANE,), jnp.float32),
        pltpu.VMEM_SHARED((PLANE,), jnp.float32),
        pltpu.VMEM((NB, NROWS, 128), jnp.int32),
        pltpu.VMEM((CHUNK,), jnp.float32),
        pltpu.VMEM((ZB,), jnp.float32),
        pltpu.VMEM((CHUNK,), jnp.int32),
        pltpu.VMEM((CHUNK,), jnp.int32),
        pltpu.VMEM((CHUNK,), jnp.int32),
        pltpu.VMEM((STRIPE_W,), jnp.float32),
        pltpu.VMEM((STRIPE_W,), jnp.float32),
        pltpu.VMEM((8, 384), jnp.float32),
        pltpu.VMEM((8, 384), jnp.float32),
        pltpu.VMEM((8, 16), jnp.float32),
        pltpu.VMEM((8, 16), jnp.float32),
        pltpu.VMEM((8, 128), jnp.float32),
        pltpu.VMEM((STRIPE_W,), jnp.float32),
        pltpu.SemaphoreType.DMA,
        pltpu.SemaphoreType.DMA,
        pltpu.SemaphoreType.DMA,
        pltpu.SemaphoreType.DMA,
        pltpu.SemaphoreType.DMA,
        pltpu.SemaphoreType.DMA,
    ],
    mesh=_mesh,
)
def _scatter_planes(vals_hbm, y_hbm, x_hbm, m_hbm, out_hbm,
                    plane_a, plane_b, idx_v, vals_v, zeros_v, y_v, x_v, m_v,
                    gf_a, gf_b, g2d_a, g2d_b, g2dt_a, g2dt_b, g2db, zeros4k,
                    sem_s, sem_f0, sem_f1, sem_e0, sem_e1, sem_z):
    planes = (plane_a, plane_b)
    g2ds = (g2d_a, g2d_b)
    g2dts = (g2dt_a, g2dt_b)
    semf = (sem_f0, sem_f1)
    seme = (sem_e0, sem_e1)
    c = lax.axis_index("c")
    s = lax.axis_index("s")
    base = s * CHUNK
    gfs = (gf_a, gf_b)

    zv = jnp.zeros((16,), jnp.float32)
    for i in range(ZB // 16):
        zeros_v[pl.ds(i * 16, 16)] = zv

    for i in range(STRIPE_W // 16):
        zeros4k[pl.ds(i * 16, 16)] = zv

    tile_words = PLANE // 16
    for p in range(2):
        for i in range(tile_words // ZB):
            pltpu.sync_copy(
                zeros_v,
                planes[p].at[pl.ds(s * tile_words + i * ZB, ZB)])

    lane = lax.iota(jnp.int32, 16)
    for bl in range(NB):
        b = c * NB + bl
        pltpu.sync_copy(y_hbm.at[pl.ds(b * NPAD + base, CHUNK)], y_v)
        pltpu.sync_copy(x_hbm.at[pl.ds(b * NPAD + base, CHUNK)], x_v)
        pltpu.sync_copy(m_hbm.at[pl.ds(b * NPAD + base, CHUNK)], m_v)
        for i in range(CHUNK // 16):
            yv = y_v[pl.ds(i * 16, 16)]
            xv = x_v[pl.ds(i * 16, 16)]
            mv = m_v[pl.ds(i * 16, 16)]
            addr = (((yv >> 3) << 12) + ((xv >> 7) << 10)
                    + ((yv & 7) << 7) + (xv & 127))
            dump = DUMP + ((lane + i * 16) & 1023)
            iv = jnp.where(mv == 1, addr, dump)
            idx_v[bl, i // 8, pl.ds((i % 8) * 16, 16)] = iv

    plsc.subcore_barrier()

    def issue_scatter(pbuf, bl, f):
        voff = (f * BATCH + (c * NB + bl)) * NPAD + base
        pltpu.sync_copy(vals_hbm.at[pl.ds(voff, CHUNK)], vals_v)
        return [
            pltpu.async_copy(vals_v.at[pl.ds(j * 128, 128)],
                             pbuf.at[idx_v.at[bl, j]], sem_s, add=True)
            for j in range(NROWS)
        ]

    def export_plane(pbuf, bl, fp):
        b = c * NB + bl
        fd = [None] * 4
        fd[0] = pltpu.async_copy(pbuf.at[pl.ds(s * STRIPE_W, STRIPE_W)],
                                 gfs[0], semf[0])
        wds = {}
        zds = []
        for i in range(3):
            ty = i * 16 + s
            if i < 2:
                nty = (i + 1) * 16 + s
                fd[i + 1] = pltpu.async_copy(
                    pbuf.at[pl.ds(nty * STRIPE_W, STRIPE_W)],
                    gfs[(i + 1) % 2], semf[(i + 1) % 2])
            else:
                fd[3] = pltpu.async_copy(
                    pbuf.at[pl.ds((192 + s) * 1024, 1024)],
                    gfs[1].at[pl.ds(0, 1024)], semf[1])
            fd[i].wait()
            zds.append(pltpu.async_copy(
                zeros4k, pbuf.at[pl.ds(ty * STRIPE_W, STRIPE_W)], sem_z))
            gf = gfs[i % 2]
            g2d = g2ds[i % 2]
            g2dt = g2dts[i % 2]
            if i >= 2:
                for d in wds.pop(i - 2):
                    d.wait()
            for r in range(8):
                for tx in range(3):
                    for u in range(8):
                        g2d[r, pl.ds(tx * 128 + u * 16, 16)] = (
                            gf[pl.ds(tx * 1024 + r * 128 + u * 16, 16)])
                g2dt[r, pl.ds(0, 16)] = gf[pl.ds(3 * 1024 + r * 128, 16)]
            yo = pl.multiple_of(ty * 8, 8)
            d1 = pltpu.async_copy(
                g2d, out_hbm.at[b, fp, pl.ds(yo, 8), pl.ds(0, 384)],
                seme[i % 2])
            d2 = pltpu.async_copy(
                g2dt, out_hbm.at[b, fp, pl.ds(yo, 8), pl.ds(384, 16)],
                seme[i % 2])
            wds[i] = (d1, d2)

        fd[3].wait()
        zds.append(pltpu.async_copy(
            zeros4k.at[pl.ds(0, 1024)],
            pbuf.at[pl.ds((192 + s) * 1024, 1024)], sem_z))
        for d in wds.pop(1):
            d.wait()

        @pl.when(s < 8)
        def _():
            gf = gfs[1]
            for r in range(8):
                for u in range(8):
                    g2db[r, pl.ds(u * 16, 16)] = gf[pl.ds(r * 128 + u * 16, 16)]
                g2dts[1][r, pl.ds(0, 16)] = gf[pl.ds(r * 128, 16)]
            txb = s & 3
            yo = pl.multiple_of((48 + (s >> 2)) * 8, 8)
            xo = pl.multiple_of(txb * 128, 128)

            @pl.when(txb < 3)
            def _():
                pltpu.async_copy(
                    g2db, out_hbm.at[b, fp, pl.ds(yo, 8), pl.ds(xo, 128)],
                    seme[1]).wait()

            @pl.when(txb == 3)
            def _():
                pltpu.async_copy(
                    g2dts[1],
                    out_hbm.at[b, fp, pl.ds(yo, 8), pl.ds(384, 16)],
                    seme[1]).wait()

        for d in wds.pop(2):
            d.wait()
        for d in zds:
            d.wait()

    for bl in range(NB):
        def pair_body(g, carry, bl=bl):
            for p in range(2):
                f = g * 2 + p
                pbuf = planes[p]
                qbuf = planes[1 - p]
                sds = issue_scatter(pbuf, bl, f)

                @pl.when(f >= 1)
                def _(qbuf=qbuf, bl=bl, f=f):
                    export_plane(qbuf, bl, f - 1)

                for d in sds:
                    d.wait()
                plsc.subcore_barrier()
            return carry

        lax.fori_loop(0, FEAT // 2, pair_body, 0)
        export_plane(planes[1], bl, FEAT - 1)
        plsc.subcore_barrier()


def kernel(pillars, coord, contains_pillars):
    batch, n_pillars, _ = pillars.shape
    pad = NPAD - n_pillars
    vals = jnp.transpose(pillars.astype(jnp.float32), (2, 0, 1))
    vals = jnp.pad(vals, ((0, 0), (0, 0), (0, pad))).reshape(-1)
    yc = jnp.pad(coord[:, :, 1].astype(jnp.int32), ((0, 0), (0, pad))).reshape(-1)
    xc = jnp.pad(coord[:, :, 2].astype(jnp.int32), ((0, 0), (0, pad))).reshape(-1)
    mc = jnp.pad(contains_pillars.astype(jnp.int32), ((0, 0), (0, pad))).reshape(-1)
    return _scatter_planes(vals, yc, xc, mc)
